# baseline (speedup 1.0000x reference)
import jax
import jax.numpy as jnp
from jax import lax
from jax.experimental import pallas as pl

CW = 5000


def _focal_kernel(ct_ref, a_ref, reg_ref, ann_ref, annt_ref, acc_ref):
    b = pl.program_id(0)
    j = pl.program_id(1)

    a = jnp.transpose(a_ref[...])
    ann = ann_ref[0]
    ax1 = a[0:1, :]
    ay1 = a[1:2, :]
    ax2 = a[2:3, :]
    ay2 = a[3:4, :]
    bx1 = ann[:, 0:1]
    by1 = ann[:, 1:2]
    bx2 = ann[:, 2:3]
    by2 = ann[:, 3:4]

    area_a = (ax2 - ax1) * (ay2 - ay1)
    area_b = (bx2 - bx1) * (by2 - by1)
    iw = jnp.maximum(jnp.minimum(ax2, bx2) - jnp.maximum(ax1, bx1), 0.0)
    ih = jnp.maximum(jnp.minimum(ay2, by2) - jnp.maximum(ay1, by1), 0.0)
    inter = iw * ih
    ua = jnp.maximum(area_a + area_b - inter, 1e-8)
    iou = inter / ua

    iou_max = jnp.max(iou, axis=0, keepdims=True)
    midx = lax.broadcasted_iota(jnp.int32, iou.shape, 0)
    argi = jnp.min(jnp.where(iou == iou_max, midx, 64), axis=0,
                   keepdims=True)
    onehot = (midx == argi).astype(jnp.float32)

    assigned = lax.dot_general(annt_ref[0], onehot,
                               (((1,), (0,)), ((), ())),
                               preferred_element_type=jnp.float32)
    gx1 = assigned[0:1, :]
    gy1 = assigned[1:2, :]
    gx2 = assigned[2:3, :]
    gy2 = assigned[3:4, :]
    kcol = lax.dot_general(onehot, ann[:, 4:5],
                           (((0,), (0,)), ((), ())),
                           preferred_element_type=jnp.float32)
    kidx = kcol.astype(jnp.int32)

    pos = iou_max > 0.5
    posf = pos.astype(jnp.float32)
    w = jnp.logical_or(iou_max < 0.4, pos).astype(jnp.float32)
    num_pos = jnp.sum(posf)

    c = jnp.clip(ct_ref[0], 1e-4, 1.0 - 1e-4)
    f_all = 0.75 * c * c * (-jnp.log1p(-c))
    ones_row = jnp.ones((1, c.shape[1]), jnp.float32)
    s_f = lax.dot_general(ones_row, f_all, (((1,), (1,)), ((), ())),
                          preferred_element_type=jnp.float32)
    cidx = lax.broadcasted_iota(jnp.int32, c.shape, 1)
    masked = jnp.where(cidx == kidx, c, 0.0)
    c_k = lax.dot_general(ones_row, masked, (((1,), (1,)), ((), ())),
                          preferred_element_type=jnp.float32)
    omc = 1.0 - c_k
    corr = 0.25 * omc * omc * (-jnp.log(c_k)) - 0.75 * c_k * c_k * (-jnp.log(omc))
    cls_u = jnp.sum(w * s_f + posf * corr)

    aw = ax2 - ax1
    ah = ay2 - ay1
    acx = ax1 + 0.5 * aw
    acy = ay1 + 0.5 * ah
    gw = gx2 - gx1
    gh = gy2 - gy1
    gcx = gx1 + 0.5 * gw
    gcy = gy1 + 0.5 * gh
    gw = jnp.maximum(gw, 1.0)
    gh = jnp.maximum(gh, 1.0)
    tdx = (gcx - acx) / aw * 10.0
    tdy = (gcy - acy) / ah * 10.0
    tdw = jnp.log(gw / aw) * 5.0
    tdh = jnp.log(gh / ah) * 5.0
    r = jnp.transpose(reg_ref[0])
    reg_s = 0.0
    for col, t in enumerate((tdx, tdy, tdw, tdh)):
        diff = jnp.abs(t - r[col:col + 1, :])
        rl = jnp.where(diff < 1.0 / 9.0, 4.5 * diff * diff, diff - 0.5 / 9.0)
        reg_s = reg_s + jnp.sum(posf * rl)

    lane = lax.broadcasted_iota(jnp.int32, (8, 128), 1)
    row = lax.broadcasted_iota(jnp.int32, (8, 128), 0)
    mine = row == b
    contrib = (jnp.where(mine & (lane == 0), cls_u, 0.0)
               + jnp.where(mine & (lane == 1), num_pos, 0.0)
               + jnp.where(mine & (lane == 2), reg_s, 0.0))
    first = jnp.logical_and(b == 0, j == 0)

    @pl.when(first)
    def _():
        acc_ref[...] = contrib

    @pl.when(jnp.logical_not(first))
    def _():
        acc_ref[...] += contrib


@jax.jit
def kernel(classifications, regressions, anchors, annotations):
    B, N, C = classifications.shape
    nc = N // CW

    ann_t = jnp.transpose(annotations, (0, 2, 1))

    acc = pl.pallas_call(
        _focal_kernel,
        grid=(B, nc),
        in_specs=[
            pl.BlockSpec((1, CW, C), lambda b, j: (b, j, 0)),
            pl.BlockSpec((CW, 4), lambda b, j: (j, 0)),
            pl.BlockSpec((1, CW, 4), lambda b, j: (b, j, 0)),
            pl.BlockSpec((1, 64, 5), lambda b, j: (b, 0, 0)),
            pl.BlockSpec((1, 5, 64), lambda b, j: (b, 0, 0)),
        ],
        out_specs=pl.BlockSpec((8, 128), lambda b, j: (0, 0)),
        out_shape=jax.ShapeDtypeStruct((8, 128), jnp.float32),
    )(classifications, anchors[0], regressions, annotations, ann_t)

    npos = acc[:, 1]
    cls_l = acc[:, 0] / jnp.maximum(npos, 1.0)
    reg_l = jnp.where(npos > 0, acc[:, 2] / (4.0 * jnp.maximum(npos, 1.0)), 0.0)
    return jnp.mean(cls_l) + jnp.mean(reg_l)


# speedup vs baseline: 1.3775x; 1.3775x over previous
import jax
import jax.numpy as jnp
from jax import lax
from jax.experimental import pallas as pl

CW = 5000


def _focal_kernel(ct_ref, a_ref, reg_ref, ann_ref, annt_ref, acc_ref):
    b = pl.program_id(0)
    j = pl.program_id(1)

    a = a_ref[0]
    ann = ann_ref[0]
    ax1 = a[0:1, :]
    ay1 = a[1:2, :]
    ax2 = a[2:3, :]
    ay2 = a[3:4, :]
    bx1 = ann[:, 0:1]
    by1 = ann[:, 1:2]
    bx2 = ann[:, 2:3]
    by2 = ann[:, 3:4]

    area_a = (ax2 - ax1) * (ay2 - ay1)
    area_b = (bx2 - bx1) * (by2 - by1)
    iw = jnp.maximum(jnp.minimum(ax2, bx2) - jnp.maximum(ax1, bx1), 0.0)
    ih = jnp.maximum(jnp.minimum(ay2, by2) - jnp.maximum(ay1, by1), 0.0)
    inter = iw * ih
    ua = jnp.maximum(area_a + area_b - inter, 1e-8)
    iou = inter / ua

    iou_max = jnp.max(iou, axis=0, keepdims=True)
    midx = lax.broadcasted_iota(jnp.int32, iou.shape, 0)
    argi = jnp.min(jnp.where(iou == iou_max, midx, 64), axis=0,
                   keepdims=True)
    onehot = (midx == argi).astype(jnp.float32)

    assigned = lax.dot_general(annt_ref[0], onehot,
                               (((1,), (0,)), ((), ())),
                               preferred_element_type=jnp.float32)
    gx1 = assigned[0:1, :]
    gy1 = assigned[1:2, :]
    gx2 = assigned[2:3, :]
    gy2 = assigned[3:4, :]
    kcol = lax.dot_general(onehot, ann[:, 4:5],
                           (((0,), (0,)), ((), ())),
                           preferred_element_type=jnp.float32)
    kidx = kcol.astype(jnp.int32)

    pos = iou_max > 0.5
    posf = pos.astype(jnp.float32)
    w = jnp.logical_or(iou_max < 0.4, pos).astype(jnp.float32)
    num_pos = jnp.sum(posf)

    c = jnp.clip(ct_ref[0], 1e-4, 1.0 - 1e-4)
    f_all = 0.75 * c * c * (-jnp.log(1.0 - c))
    ones_row = jnp.ones((1, c.shape[1]), jnp.float32)
    s_f = lax.dot_general(ones_row, f_all, (((1,), (1,)), ((), ())),
                          preferred_element_type=jnp.float32)
    cidx = lax.broadcasted_iota(jnp.int32, c.shape, 1)
    masked = jnp.where(cidx == kidx, c, 0.0)
    c_k = lax.dot_general(ones_row, masked, (((1,), (1,)), ((), ())),
                          preferred_element_type=jnp.float32)
    omc = 1.0 - c_k
    corr = 0.25 * omc * omc * (-jnp.log(c_k)) - 0.75 * c_k * c_k * (-jnp.log(omc))
    cls_u = jnp.sum(w * s_f + posf * corr)

    aw = ax2 - ax1
    ah = ay2 - ay1
    acx = ax1 + 0.5 * aw
    acy = ay1 + 0.5 * ah
    gw = gx2 - gx1
    gh = gy2 - gy1
    gcx = gx1 + 0.5 * gw
    gcy = gy1 + 0.5 * gh
    gw = jnp.maximum(gw, 1.0)
    gh = jnp.maximum(gh, 1.0)
    tdx = (gcx - acx) / aw * 10.0
    tdy = (gcy - acy) / ah * 10.0
    tdw = jnp.log(gw / aw) * 5.0
    tdh = jnp.log(gh / ah) * 5.0
    r = reg_ref[0, 0]
    reg_s = 0.0
    for col, t in enumerate((tdx, tdy, tdw, tdh)):
        diff = jnp.abs(t - r[col:col + 1, :])
        rl = jnp.where(diff < 1.0 / 9.0, 4.5 * diff * diff, diff - 0.5 / 9.0)
        reg_s = reg_s + jnp.sum(posf * rl)

    lane = lax.broadcasted_iota(jnp.int32, (8, 128), 1)
    row = lax.broadcasted_iota(jnp.int32, (8, 128), 0)
    mine = row == b
    contrib = (jnp.where(mine & (lane == 0), cls_u, 0.0)
               + jnp.where(mine & (lane == 1), num_pos, 0.0)
               + jnp.where(mine & (lane == 2), reg_s, 0.0))
    first = jnp.logical_and(b == 0, j == 0)

    @pl.when(first)
    def _():
        acc_ref[...] = contrib

    @pl.when(jnp.logical_not(first))
    def _():
        acc_ref[...] += contrib


@jax.jit
def kernel(classifications, regressions, anchors, annotations):
    B, N, C = classifications.shape
    nc = N // CW

    a_c = anchors[0].T.reshape(4, nc, CW).transpose(1, 0, 2)
    reg_c = regressions.transpose(0, 2, 1).reshape(B, 4, nc, CW)
    reg_c = reg_c.transpose(0, 2, 1, 3)
    ann_t = jnp.transpose(annotations, (0, 2, 1))

    acc = pl.pallas_call(
        _focal_kernel,
        grid=(B, nc),
        in_specs=[
            pl.BlockSpec((1, CW, C), lambda b, j: (b, j, 0)),
            pl.BlockSpec((1, 4, CW), lambda b, j: (j, 0, 0)),
            pl.BlockSpec((1, 1, 4, CW), lambda b, j: (b, j, 0, 0)),
            pl.BlockSpec((1, 64, 5), lambda b, j: (b, 0, 0)),
            pl.BlockSpec((1, 5, 64), lambda b, j: (b, 0, 0)),
        ],
        out_specs=pl.BlockSpec((8, 128), lambda b, j: (0, 0)),
        out_shape=jax.ShapeDtypeStruct((8, 128), jnp.float32),
    )(classifications, a_c, reg_c, annotations, ann_t)

    npos = acc[:, 1]
    cls_l = acc[:, 0] / jnp.maximum(npos, 1.0)
    reg_l = jnp.where(npos > 0, acc[:, 2] / (4.0 * jnp.maximum(npos, 1.0)), 0.0)
    return jnp.mean(cls_l) + jnp.mean(reg_l)


# speedup vs baseline: 1.4713x; 1.0681x over previous
import jax
import jax.numpy as jnp
from jax import lax
from jax.experimental import pallas as pl

CW = 10000


def _focal_kernel(ct_ref, a_ref, reg_ref, ann_ref, annt_ref, acc_ref):
    b = pl.program_id(0)
    j = pl.program_id(1)

    a = a_ref[0]
    ann = ann_ref[0]
    ax1 = a[0:1, :]
    ay1 = a[1:2, :]
    ax2 = a[2:3, :]
    ay2 = a[3:4, :]
    bx1 = ann[:, 0:1]
    by1 = ann[:, 1:2]
    bx2 = ann[:, 2:3]
    by2 = ann[:, 3:4]

    area_a = (ax2 - ax1) * (ay2 - ay1)
    area_b = (bx2 - bx1) * (by2 - by1)
    iw = jnp.maximum(jnp.minimum(ax2, bx2) - jnp.maximum(ax1, bx1), 0.0)
    ih = jnp.maximum(jnp.minimum(ay2, by2) - jnp.maximum(ay1, by1), 0.0)
    inter = iw * ih
    ua = jnp.maximum(area_a + area_b - inter, 1e-8)
    iou = inter / ua

    iou_max = jnp.max(iou, axis=0, keepdims=True)
    midx = lax.broadcasted_iota(jnp.int32, iou.shape, 0)
    argi = jnp.min(jnp.where(iou == iou_max, midx, 64), axis=0,
                   keepdims=True)
    onehot = (midx == argi).astype(jnp.float32)

    assigned = lax.dot_general(annt_ref[0], onehot,
                               (((1,), (0,)), ((), ())),
                               preferred_element_type=jnp.float32)
    gx1 = assigned[0:1, :]
    gy1 = assigned[1:2, :]
    gx2 = assigned[2:3, :]
    gy2 = assigned[3:4, :]
    kcol = lax.dot_general(onehot, ann[:, 4:5],
                           (((0,), (0,)), ((), ())),
                           preferred_element_type=jnp.float32)
    kidx = kcol.astype(jnp.int32)

    pos = iou_max > 0.5
    posf = pos.astype(jnp.float32)
    w = jnp.logical_or(iou_max < 0.4, pos).astype(jnp.float32)
    num_pos = jnp.sum(posf)

    c = jnp.clip(ct_ref[0], 1e-4, 1.0 - 1e-4)
    f_all = 0.75 * c * c * (-jnp.log(1.0 - c))
    ones_row = jnp.ones((1, c.shape[1]), jnp.float32)
    s_f = lax.dot_general(ones_row, f_all, (((1,), (1,)), ((), ())),
                          preferred_element_type=jnp.float32)
    cidx = lax.broadcasted_iota(jnp.int32, c.shape, 1)
    masked = jnp.where(cidx == kidx, c, 0.0)
    c_k = lax.dot_general(ones_row, masked, (((1,), (1,)), ((), ())),
                          preferred_element_type=jnp.float32)
    omc = 1.0 - c_k
    corr = 0.25 * omc * omc * (-jnp.log(c_k)) - 0.75 * c_k * c_k * (-jnp.log(omc))
    cls_u = jnp.sum(w * s_f + posf * corr)

    aw = ax2 - ax1
    ah = ay2 - ay1
    acx = ax1 + 0.5 * aw
    acy = ay1 + 0.5 * ah
    gw = gx2 - gx1
    gh = gy2 - gy1
    gcx = gx1 + 0.5 * gw
    gcy = gy1 + 0.5 * gh
    gw = jnp.maximum(gw, 1.0)
    gh = jnp.maximum(gh, 1.0)
    tdx = (gcx - acx) / aw * 10.0
    tdy = (gcy - acy) / ah * 10.0
    tdw = jnp.log(gw / aw) * 5.0
    tdh = jnp.log(gh / ah) * 5.0
    r = reg_ref[0, 0]
    reg_s = 0.0
    for col, t in enumerate((tdx, tdy, tdw, tdh)):
        diff = jnp.abs(t - r[col:col + 1, :])
        rl = jnp.where(diff < 1.0 / 9.0, 4.5 * diff * diff, diff - 0.5 / 9.0)
        reg_s = reg_s + jnp.sum(posf * rl)

    lane = lax.broadcasted_iota(jnp.int32, (8, 128), 1)
    row = lax.broadcasted_iota(jnp.int32, (8, 128), 0)
    mine = row == b
    contrib = (jnp.where(mine & (lane == 0), cls_u, 0.0)
               + jnp.where(mine & (lane == 1), num_pos, 0.0)
               + jnp.where(mine & (lane == 2), reg_s, 0.0))
    first = jnp.logical_and(b == 0, j == 0)

    @pl.when(first)
    def _():
        acc_ref[...] = contrib

    @pl.when(jnp.logical_not(first))
    def _():
        acc_ref[...] += contrib


@jax.jit
def kernel(classifications, regressions, anchors, annotations):
    B, N, C = classifications.shape
    nc = N // CW

    a_c = anchors[0].T.reshape(4, nc, CW).transpose(1, 0, 2)
    reg_c = regressions.transpose(0, 2, 1).reshape(B, 4, nc, CW)
    reg_c = reg_c.transpose(0, 2, 1, 3)
    ann_t = jnp.transpose(annotations, (0, 2, 1))

    acc = pl.pallas_call(
        _focal_kernel,
        grid=(B, nc),
        in_specs=[
            pl.BlockSpec((1, CW, C), lambda b, j: (b, j, 0)),
            pl.BlockSpec((1, 4, CW), lambda b, j: (j, 0, 0)),
            pl.BlockSpec((1, 1, 4, CW), lambda b, j: (b, j, 0, 0)),
            pl.BlockSpec((1, 64, 5), lambda b, j: (b, 0, 0)),
            pl.BlockSpec((1, 5, 64), lambda b, j: (b, 0, 0)),
        ],
        out_specs=pl.BlockSpec((8, 128), lambda b, j: (0, 0)),
        out_shape=jax.ShapeDtypeStruct((8, 128), jnp.float32),
    )(classifications, a_c, reg_c, annotations, ann_t)

    npos = acc[:, 1]
    cls_l = acc[:, 0] / jnp.maximum(npos, 1.0)
    reg_l = jnp.where(npos > 0, acc[:, 2] / (4.0 * jnp.maximum(npos, 1.0)), 0.0)
    return jnp.mean(cls_l) + jnp.mean(reg_l)


# speedup vs baseline: 1.4780x; 1.0045x over previous
import jax
import jax.numpy as jnp
from jax import lax
from jax.experimental import pallas as pl

CW = 25000


def _focal_kernel(ct_ref, a_ref, reg_ref, ann_ref, annt_ref, acc_ref):
    b = pl.program_id(0)
    j = pl.program_id(1)

    a = a_ref[0]
    ann = ann_ref[0]
    ax1 = a[0:1, :]
    ay1 = a[1:2, :]
    ax2 = a[2:3, :]
    ay2 = a[3:4, :]
    bx1 = ann[:, 0:1]
    by1 = ann[:, 1:2]
    bx2 = ann[:, 2:3]
    by2 = ann[:, 3:4]

    area_a = (ax2 - ax1) * (ay2 - ay1)
    area_b = (bx2 - bx1) * (by2 - by1)
    iw = jnp.maximum(jnp.minimum(ax2, bx2) - jnp.maximum(ax1, bx1), 0.0)
    ih = jnp.maximum(jnp.minimum(ay2, by2) - jnp.maximum(ay1, by1), 0.0)
    inter = iw * ih
    ua = jnp.maximum(area_a + area_b - inter, 1e-8)
    iou = inter / ua

    iou_max = jnp.max(iou, axis=0, keepdims=True)
    midx = lax.broadcasted_iota(jnp.int32, iou.shape, 0)
    argi = jnp.min(jnp.where(iou == iou_max, midx, 64), axis=0,
                   keepdims=True)
    onehot = (midx == argi).astype(jnp.float32)

    assigned = lax.dot_general(annt_ref[0], onehot,
                               (((1,), (0,)), ((), ())),
                               preferred_element_type=jnp.float32)
    gx1 = assigned[0:1, :]
    gy1 = assigned[1:2, :]
    gx2 = assigned[2:3, :]
    gy2 = assigned[3:4, :]
    kcol = lax.dot_general(onehot, ann[:, 4:5],
                           (((0,), (0,)), ((), ())),
                           preferred_element_type=jnp.float32)
    kidx = kcol.astype(jnp.int32)

    pos = iou_max > 0.5
    posf = pos.astype(jnp.float32)
    w = jnp.logical_or(iou_max < 0.4, pos).astype(jnp.float32)
    num_pos = jnp.sum(posf)

    c = jnp.clip(ct_ref[0], 1e-4, 1.0 - 1e-4)
    f_all = 0.75 * c * c * (-jnp.log(1.0 - c))
    ones_row = jnp.ones((1, c.shape[1]), jnp.float32)
    s_f = lax.dot_general(ones_row, f_all, (((1,), (1,)), ((), ())),
                          preferred_element_type=jnp.float32)
    cidx = lax.broadcasted_iota(jnp.int32, c.shape, 1)
    masked = jnp.where(cidx == kidx, c, 0.0)
    c_k = lax.dot_general(ones_row, masked, (((1,), (1,)), ((), ())),
                          preferred_element_type=jnp.float32)
    omc = 1.0 - c_k
    corr = 0.25 * omc * omc * (-jnp.log(c_k)) - 0.75 * c_k * c_k * (-jnp.log(omc))
    cls_u = jnp.sum(w * s_f + posf * corr)

    aw = ax2 - ax1
    ah = ay2 - ay1
    acx = ax1 + 0.5 * aw
    acy = ay1 + 0.5 * ah
    gw = gx2 - gx1
    gh = gy2 - gy1
    gcx = gx1 + 0.5 * gw
    gcy = gy1 + 0.5 * gh
    gw = jnp.maximum(gw, 1.0)
    gh = jnp.maximum(gh, 1.0)
    tdx = (gcx - acx) / aw * 10.0
    tdy = (gcy - acy) / ah * 10.0
    tdw = jnp.log(gw / aw) * 5.0
    tdh = jnp.log(gh / ah) * 5.0
    r = reg_ref[0, 0]
    rl_vec = 0.0
    for col, t in enumerate((tdx, tdy, tdw, tdh)):
        diff = jnp.abs(t - r[col:col + 1, :])
        rl_vec = rl_vec + jnp.where(diff < 1.0 / 9.0, 4.5 * diff * diff,
                                    diff - 0.5 / 9.0)
    reg_s = jnp.sum(posf * rl_vec)

    lane = lax.broadcasted_iota(jnp.int32, (8, 128), 1)
    row = lax.broadcasted_iota(jnp.int32, (8, 128), 0)
    mine = row == b
    contrib = (jnp.where(mine & (lane == 0), cls_u, 0.0)
               + jnp.where(mine & (lane == 1), num_pos, 0.0)
               + jnp.where(mine & (lane == 2), reg_s, 0.0))
    first = jnp.logical_and(b == 0, j == 0)

    @pl.when(first)
    def _():
        acc_ref[...] = contrib

    @pl.when(jnp.logical_not(first))
    def _():
        acc_ref[...] += contrib


@jax.jit
def kernel(classifications, regressions, anchors, annotations):
    B, N, C = classifications.shape
    nc = N // CW

    a_c = anchors[0].T.reshape(4, nc, CW).transpose(1, 0, 2)
    reg_c = regressions.transpose(0, 2, 1).reshape(B, 4, nc, CW)
    reg_c = reg_c.transpose(0, 2, 1, 3)
    ann_t = jnp.transpose(annotations, (0, 2, 1))

    acc = pl.pallas_call(
        _focal_kernel,
        grid=(B, nc),
        in_specs=[
            pl.BlockSpec((1, CW, C), lambda b, j: (b, j, 0)),
            pl.BlockSpec((1, 4, CW), lambda b, j: (j, 0, 0)),
            pl.BlockSpec((1, 1, 4, CW), lambda b, j: (b, j, 0, 0)),
            pl.BlockSpec((1, 64, 5), lambda b, j: (b, 0, 0)),
            pl.BlockSpec((1, 5, 64), lambda b, j: (b, 0, 0)),
        ],
        out_specs=pl.BlockSpec((8, 128), lambda b, j: (0, 0)),
        out_shape=jax.ShapeDtypeStruct((8, 128), jnp.float32),
    )(classifications, a_c, reg_c, annotations, ann_t)

    npos = acc[:, 1]
    cls_l = acc[:, 0] / jnp.maximum(npos, 1.0)
    reg_l = jnp.where(npos > 0, acc[:, 2] / (4.0 * jnp.maximum(npos, 1.0)), 0.0)
    return jnp.mean(cls_l) + jnp.mean(reg_l)
